# baseline (speedup 1.0000x reference)
import functools
import math

import jax
import jax.numpy as jnp
from jax import lax
from jax.experimental import pallas as pl
from jax.experimental.pallas import tpu as pltpu
from jax.experimental.pallas import tpu_sc as plsc

D_MODEL = 64
SCALE = math.sqrt(D_MODEL)

_NC = 2
_NS = 16
_NW = _NC * _NS
_LANES = 16
_CHUNK = 512


def _make_gather_kernel(n_rows: int):
    assert n_rows % (_NW * _CHUNK) == 0
    rows_per_w = n_rows // _NW
    n_chunks = rows_per_w // _CHUNK
    vecs_per_chunk = _CHUNK * D_MODEL // _LANES

    mesh = plsc.VectorSubcoreMesh(core_axis_name="c", subcore_axis_name="s")

    @functools.partial(
        pl.kernel,
        out_type=jax.ShapeDtypeStruct((n_rows, D_MODEL), jnp.float32),
        mesh=mesh,
        scratch_types=[
            pltpu.VMEM((_CHUNK,), jnp.int32),
            pltpu.VMEM((_CHUNK, D_MODEL), jnp.float32),
            pltpu.SemaphoreType.DMA,
        ],
        compiler_params=pltpu.CompilerParams(use_tc_tiling_on_sc=False),
    )
    def gather_scale(lut_hbm, idx_hbm, out_hbm, idx_v, rows_v, sem):
        wid = lax.axis_index("s") * _NC + lax.axis_index("c")
        base = wid * rows_per_w

        @pl.loop(0, n_chunks)
        def _chunks(g):
            off = base + g * _CHUNK
            pltpu.sync_copy(idx_hbm.at[pl.ds(off, _CHUNK)], idx_v)
            pltpu.async_copy(lut_hbm.at[idx_v], rows_v, sem).wait()

            @pl.loop(0, _CHUNK, unroll=4)
            def _scale(r):
                for d in range(D_MODEL // _LANES):
                    sl = pl.ds(d * _LANES, _LANES)
                    rows_v[r, sl] = rows_v[r, sl] * SCALE

            pltpu.sync_copy(rows_v, out_hbm.at[pl.ds(off, _CHUNK)])

    return gather_scale


def kernel(x, lut):
    b, s = x.shape
    flat_idx = x.reshape(b * s).astype(jnp.int32)
    out = _make_gather_kernel(b * s)(lut, flat_idx)
    return out.reshape(b, s, D_MODEL)


# speedup vs baseline: 1.0769x; 1.0769x over previous
import functools
import math

import jax
import jax.numpy as jnp
from jax import lax
from jax.experimental import pallas as pl
from jax.experimental.pallas import tpu as pltpu
from jax.experimental.pallas import tpu_sc as plsc

D_MODEL = 64
SCALE = math.sqrt(D_MODEL)

_NC = 2
_NS = 16
_NW = _NC * _NS
_LANES = 16
_CHUNK = 400
_NBUF = 4


def _make_gather_kernel(n_rows: int):
    assert n_rows % (_NW * _CHUNK * _NBUF) == 0
    rows_per_w = n_rows // _NW
    n_outer = rows_per_w // (_CHUNK * _NBUF)

    mesh = plsc.VectorSubcoreMesh(core_axis_name="c", subcore_axis_name="s")

    scratch = (
        [pltpu.VMEM((_CHUNK,), jnp.int32) for _ in range(_NBUF)]
        + [pltpu.VMEM((_CHUNK, D_MODEL), jnp.float32) for _ in range(_NBUF)]
        + [pltpu.SemaphoreType.DMA] * (3 * _NBUF)
    )

    @functools.partial(
        pl.kernel,
        out_type=jax.ShapeDtypeStruct((n_rows, D_MODEL), jnp.float32),
        mesh=mesh,
        scratch_types=scratch,
        compiler_params=pltpu.CompilerParams(use_tc_tiling_on_sc=False),
    )
    def gather_scale(lut_hbm, idx_hbm, out_hbm, *sc):
        idx_v = sc[:_NBUF]
        rows_v = sc[_NBUF:2 * _NBUF]
        idx_s = sc[2 * _NBUF:3 * _NBUF]
        in_s = sc[3 * _NBUF:4 * _NBUF]
        out_s = sc[4 * _NBUF:5 * _NBUF]

        wid = lax.axis_index("s") * _NC + lax.axis_index("c")
        base = wid * rows_per_w

        @pl.loop(0, n_outer)
        def _outer(o):
            goff = base + o * (_CHUNK * _NBUF)

            idx_dma = [
                pltpu.async_copy(
                    idx_hbm.at[pl.ds(goff + b * _CHUNK, _CHUNK)],
                    idx_v[b], idx_s[b])
                for b in range(_NBUF)
            ]

            gather_dma = []
            for b in range(_NBUF):
                @pl.when(o > 0)
                def _drain():
                    pltpu.make_async_copy(
                        rows_v[b],
                        out_hbm.at[pl.ds(0, _CHUNK)],
                        out_s[b]).wait()
                idx_dma[b].wait()
                gather_dma.append(
                    pltpu.async_copy(lut_hbm.at[idx_v[b]], rows_v[b],
                                     in_s[b]))

            for b in range(_NBUF):
                gather_dma[b].wait()

                @pl.loop(0, _CHUNK, unroll=8)
                def _scale(r):
                    for d in range(D_MODEL // _LANES):
                        sl = pl.ds(d * _LANES, _LANES)
                        rows_v[b][r, sl] = rows_v[b][r, sl] * SCALE

                pltpu.async_copy(
                    rows_v[b],
                    out_hbm.at[pl.ds(goff + b * _CHUNK, _CHUNK)],
                    out_s[b])

        for b in range(_NBUF):
            pltpu.make_async_copy(
                rows_v[b], out_hbm.at[pl.ds(0, _CHUNK)], out_s[b]).wait()

    return gather_scale


def kernel(x, lut):
    b, s = x.shape
    flat_idx = x.reshape(b * s).astype(jnp.int32)
    out = _make_gather_kernel(b * s)(lut, flat_idx)
    return out.reshape(b, s, D_MODEL)
